# baseline (speedup 1.0000x reference)
import functools

import jax
import jax.numpy as jnp
from jax import lax
from jax.experimental import pallas as pl
from jax.experimental.pallas import tpu as pltpu
from jax.experimental.pallas import tpu_sc as plsc

GAMMA = 0.99

B, L, V, F = 4, 64, 1024, 64
ROWS = B * L
NLANE = 16
NW = 16
RPW = ROWS // NW
PAIR_W = 2 * F

_DNUMS = lax.GatherDimensionNumbers(
    offset_dims=(), collapsed_slice_dims=(0,), start_index_map=(0,))


def _gather16(vec, idx):
    return lax.gather(vec, idx[:, None], _DNUMS, (1,),
                      mode=lax.GatherScatterMode.PROMISE_IN_BOUNDS)


def _allsum16(v):
    lanes = lax.iota(jnp.int32, NLANE)
    for s in (1, 2, 4, 8):
        v = v + _gather16(v, lanes ^ s)
    return v


def _sc_body(psi_hbm, tgt_hbm, act_hbm, sl_hbm, feat_hbm, out_hbm,
             act_v, sl_v, pidx_v, tidx_v, fidx_v,
             psi_rows, tgt_rows, phi_rows, acc_v, iz_v, sums_v, out_v, shared,
             sem):
    c = lax.axis_index("c")
    s = lax.axis_index("s")

    @pl.when(c == 0)
    def _():
        lanes = lax.iota(jnp.int32, NLANE)
        base = s * RPW
        b = base // L

        pltpu.sync_copy(act_hbm.at[pl.ds(base, 2 * NLANE)], act_v)
        pltpu.sync_copy(sl_hbm, sl_v)

        a_cur = act_v[pl.ds(0, NLANE)]
        a_hi = act_v[pl.ds(NLANE, NLANE)]
        p_vec = lanes + base
        q_vec = jnp.minimum(p_vec + 1, ROWS - 1)
        ql = q_vec - base
        is_hi = jnp.minimum(jnp.maximum(ql - (NLANE - 1), 0), 1)
        g_lo = _gather16(a_cur, jnp.minimum(ql, NLANE - 1))
        g_hi = _gather16(a_hi, jnp.maximum(ql - NLANE, 0))
        a_nxt = g_lo + is_hi * (g_hi - g_lo)

        pidx_v[...] = p_vec * (V // 2) + jnp.right_shift(a_cur, 1)
        tidx_v[...] = q_vec * (V // 2) + jnp.right_shift(a_nxt, 1)
        fidx = jnp.minimum(lanes, B - 1) * (L + 1) + sl_v[...]
        fidx_v[...] = jnp.right_shift(fidx, 1)
        h_cur = (a_cur & 1).astype(jnp.float32)
        h_nxt = (a_nxt & 1).astype(jnp.float32)
        h_phi = (fidx & 1).astype(jnp.float32)

        cp = pltpu.async_copy(psi_hbm.at[pidx_v], psi_rows, sem)
        ct = pltpu.async_copy(tgt_hbm.at[tidx_v], tgt_rows, sem)
        cf = pltpu.async_copy(feat_hbm.at[fidx_v], phi_rows, sem)
        cp.wait()
        ct.wait()
        cf.wait()

        gamma = jnp.float32(GAMMA)
        one = jnp.float32(1.0)
        zero = jnp.float32(0.0)
        bb = jnp.full((NLANE,), b, jnp.int32)
        d0 = _gather16(sl_v[...], bb) - 1 - (base - b * L)
        hb = _gather16(h_phi, bb)
        phi_cs = []
        for cc in range(F // NLANE):
            off = cc * NLANE
            phi_l = phi_rows[b, pl.ds(off, NLANE)]
            phi_r = phi_rows[b, pl.ds(F + off, NLANE)]
            phi_cs.append(phi_l + hb * (phi_r - phi_l))

        acc = jnp.zeros((NLANE,), jnp.float32)
        for j in range(RPW):
            row = base + j
            t = jnp.bitwise_and(row, L - 1)
            jj = jnp.full((NLANE,), j, jnp.int32)
            h = _gather16(h_cur, jj)
            hn = _gather16(h_nxt, jj)
            m = one - jnp.minimum(jnp.abs((d0 - jj).astype(jnp.float32)), one)
            w_shift = jnp.where(t <= L - 3, gamma, zero)
            w_valid = jnp.where(t <= L - 2, one, zero)
            w_tgt = (one - m) * w_shift
            for cc in range(F // NLANE):
                off = cc * NLANE
                psi_l = psi_rows[j, pl.ds(off, NLANE)]
                psi_r = psi_rows[j, pl.ds(F + off, NLANE)]
                psi_c = psi_l + h * (psi_r - psi_l)
                tgt_l = tgt_rows[j, pl.ds(off, NLANE)]
                tgt_r = tgt_rows[j, pl.ds(F + off, NLANE)]
                tgt_c = tgt_l + hn * (tgt_r - tgt_l)
                diff = psi_c - (m * phi_cs[cc] + w_tgt * tgt_c)
                acc = acc + w_valid * (diff * diff)

        @pl.when(s == 0)
        def _():
            acc_v[0, ...] = jnp.zeros((NLANE,), jnp.float32)
            pltpu.sync_copy(acc_v, shared)
        plsc.subcore_barrier()
        acc_v[0, ...] = acc
        iz_v[...] = jnp.zeros((1,), jnp.int32)
        pltpu.sync_copy(acc_v, shared.at[iz_v], add=True)
        plsc.subcore_barrier()

        @pl.when(s == 0)
        def _():
            pltpu.sync_copy(shared, sums_v)
            total = _allsum16(sums_v[0, pl.ds(0, NLANE)])
            denom = _allsum16(sl_v[...].astype(jnp.float32))
            out_v[...] = total / denom
            pltpu.sync_copy(out_v, out_hbm)


_sarsa_sc = functools.partial(
    pl.kernel,
    mesh=plsc.VectorSubcoreMesh(core_axis_name="c", subcore_axis_name="s"),
    out_type=jax.ShapeDtypeStruct((NLANE,), jnp.float32),
    scratch_types=[
        pltpu.VMEM((2 * NLANE,), jnp.int32),
        pltpu.VMEM((NLANE,), jnp.int32),
        pltpu.VMEM((NLANE,), jnp.int32),
        pltpu.VMEM((NLANE,), jnp.int32),
        pltpu.VMEM((NLANE,), jnp.int32),
        pltpu.VMEM((RPW, PAIR_W), jnp.float32),
        pltpu.VMEM((RPW, PAIR_W), jnp.float32),
        pltpu.VMEM((NLANE, PAIR_W), jnp.float32),
        pltpu.VMEM((1, NLANE), jnp.float32),
        pltpu.VMEM((1,), jnp.int32),
        pltpu.VMEM((1, NLANE), jnp.float32),
        pltpu.VMEM((NLANE,), jnp.float32),
        pltpu.VMEM_SHARED((1, NLANE), jnp.float32),
        pltpu.SemaphoreType.DMA,
    ],
)(_sc_body)


def kernel(psi, target_psi, actions, features, seq_lens):
    psi_pairs = psi.reshape(B * L * V * F // PAIR_W, PAIR_W)
    tgt_pairs = target_psi.reshape(B * L * V * F // PAIR_W, PAIR_W)
    act = jnp.pad(actions.astype(jnp.int32), ((0, 0), (0, 1))).reshape(-1)
    act = jnp.pad(act, (0, NLANE))
    sl = jnp.zeros((NLANE,), jnp.int32).at[:B].set(seq_lens.astype(jnp.int32))
    feat_pairs = features.reshape(B * (L + 1) * F // PAIR_W, PAIR_W)
    out = _sarsa_sc(psi_pairs, tgt_pairs, act, sl, feat_pairs)
    return out[0]
